# baseline (speedup 1.0000x reference)
import functools

import jax
import jax.numpy as jnp
from jax import lax
from jax.experimental import pallas as pl
from jax.experimental.pallas import tpu as pltpu
from jax.experimental.pallas import tpu_sc as plsc

NN = 10000
EE = 320000
DH = 128
FW = 64
NPAD = 10112
NCORES = 2
NSUB = 16
NW = NCORES * NSUB
CHUNK = 128
CPW = 160
EPW = CPW * CHUNK
EP = NSUB * EPW
EPT = EE // NSUB
F32 = jnp.float32

_mesh = plsc.VectorSubcoreMesh(core_axis_name="c", subcore_axis_name="s")


NNP = 10240
RPT = NNP // NSUB


def _hist_combine(hist_v, slots, acc_v, tmp_v, ci, si):
    base = si * RPT

    @pl.when(ci == 0)
    def _():
        pltpu.sync_copy(hist_v, slots.at[si])
    plsc.subcore_barrier()

    @pl.when(ci == 0)
    def _():
        for t in range(NSUB):
            pltpu.sync_copy(slots.at[t].at[pl.ds(base, RPT)], tmp_v)
            if t == 0:
                @pl.loop(0, RPT // 16)
                def _(i):
                    acc_v[pl.ds(i * 16, 16)] = tmp_v[pl.ds(i * 16, 16)]
            else:
                @pl.loop(0, RPT // 16)
                def _(i):
                    acc_v[pl.ds(i * 16, 16)] = (acc_v[pl.ds(i * 16, 16)]
                                                + tmp_v[pl.ds(i * 16, 16)])


def _make_agg_body(mode):
    def body(hl_hbm, hr_hbm, src_hbm, dst_hbm, z_hbm, inv_hbm,
             out_hbm, stat_out,
             src_v, dst_v, rows0, rows1, rows2, rows3, hist_v, stat_v,
             acc_v, tmp_v, acc, slots,
             semg0, semg1, semg2, semg3, sems0, sems1, sems2, sems3):
        ci = lax.axis_index("c")
        si = lax.axis_index("s")
        rows_per_tile = NPAD // NSUB
        base = si * rows_per_tile
        zeros16 = jnp.zeros((16,), F32)
        ones16 = jnp.ones((16,), F32)

        pltpu.sync_copy(src_hbm.at[si], src_v)
        pltpu.sync_copy(dst_hbm.at[si], dst_v)
        pltpu.sync_copy(z_hbm, acc.at[pl.ds(base, rows_per_tile)])

        @pl.when(ci == 0)
        def _():
            if mode == "c":
                pltpu.sync_copy(inv_hbm, stat_v)
                @pl.loop(0, (NNP - NN) // 16)
                def _(i):
                    stat_v[pl.ds(NN + i * 16, 16)] = zeros16

            @pl.loop(0, NNP // 16)
            def _(i):
                hist_v[pl.ds(i * 16, 16)] = zeros16

        plsc.subcore_barrier()

        def stat_chunk(k):
            for m in range(CHUNK // 16):
                d16 = dst_v[k, pl.ds(m * 16, 16)]
                if mode == "deg":
                    plsc.addupdate_scatter(hist_v, [d16], ones16)
                else:
                    s16 = src_v[k, pl.ds(m * 16, 16)]
                    g = plsc.load_gather(stat_v, [d16])
                    plsc.addupdate_scatter(hist_v, [s16], g)

        bufs = [rows0, rows1, rows2, rows3]
        gsems = [semg0, semg1, semg2, semg3]
        ssems = [sems0, sems1, sems2, sems3]
        nbuf = 2

        def pipeline(h_hbm, with_stats):
            for b in range(nbuf):
                pltpu.async_copy(h_hbm.at[src_v.at[b]], bufs[b], gsems[b])

            @pl.loop(0, CPW, step=nbuf)
            def _(j):
                for b in range(nbuf):
                    k = j + b
                    if with_stats:
                        stat_chunk(k)
                    pltpu.make_async_copy(h_hbm.at[src_v.at[0]], bufs[b],
                                          gsems[b]).wait()
                    pltpu.async_copy(bufs[b], acc.at[dst_v.at[k]], ssems[b],
                                     add=True)
                for b in range(nbuf):
                    k = j + b
                    pltpu.make_async_copy(bufs[b], acc.at[dst_v.at[0]],
                                          ssems[b]).wait()

                    @pl.when(k + nbuf < CPW)
                    def _():
                        pltpu.async_copy(h_hbm.at[src_v.at[k + nbuf]],
                                         bufs[b], gsems[b])

        @pl.when(ci == 0)
        def _():
            pipeline(hl_hbm, True)

        @pl.when(ci == 1)
        def _():
            pipeline(hr_hbm, False)

        plsc.subcore_barrier()
        pltpu.sync_copy(acc.at[pl.ds(base, rows_per_tile)],
                        out_hbm.at[ci].at[pl.ds(base, rows_per_tile)])

        _hist_combine(hist_v, slots, acc_v, tmp_v, ci, si)

        @pl.when(ci == 0)
        def _():
            if mode == "deg":
                @pl.loop(0, RPT // 16)
                def _(i):
                    d = acc_v[pl.ds(i * 16, 16)]
                    acc_v[pl.ds(i * 16, 16)] = 1.0 / jnp.maximum(d, 1.0)

            pltpu.sync_copy(acc_v, stat_out.at[pl.ds(si * RPT, RPT)])

    return body


def _make_agg_call(mode):
    return pl.kernel(
        _make_agg_body(mode),
        out_type=(jax.ShapeDtypeStruct((NCORES, NPAD, FW), F32),
                  jax.ShapeDtypeStruct((NNP,), F32)),
        mesh=_mesh,
        scratch_types=[
            pltpu.VMEM((CPW, CHUNK), jnp.int32),
            pltpu.VMEM((CPW, CHUNK), jnp.int32),
            pltpu.VMEM((CHUNK, FW), F32),
            pltpu.VMEM((CHUNK, FW), F32),
            pltpu.VMEM((CHUNK, FW), F32),
            pltpu.VMEM((CHUNK, FW), F32),
            pltpu.VMEM((NNP,), F32),
            pltpu.VMEM((NNP,), F32),
            pltpu.VMEM((RPT,), F32),
            pltpu.VMEM((RPT,), F32),
            pltpu.VMEM_SHARED((NPAD, FW), F32),
            pltpu.VMEM_SHARED((NSUB, NNP), F32),
        ] + [pltpu.SemaphoreType.DMA] * 8,
        compiler_params=pltpu.CompilerParams(needs_layout_passes=False,
                                             use_tc_tiling_on_sc=False),
    )


_agg_deg_call = _make_agg_call("deg")
_agg_c_call = _make_agg_call("c")


def _dense_body(hl_ref, hr_ref, p_ref, inv_ref, ws_ref, wn_ref,
                b_ref, ol_ref, or_ref):
    inv = inv_ref[...]
    agg_l = p_ref[0] * inv
    agg_r = p_ref[1] * inv
    ws = ws_ref[...]
    wn = wn_ref[...]
    out = (jnp.dot(hl_ref[...], ws[0:FW, :], preferred_element_type=F32)
           + jnp.dot(hr_ref[...], ws[FW:DH, :], preferred_element_type=F32)
           + jnp.dot(agg_l, wn[0:FW, :], preferred_element_type=F32)
           + jnp.dot(agg_r, wn[FW:DH, :], preferred_element_type=F32)
           + b_ref[...])
    out = jnp.maximum(out, 0.0)
    ol_ref[...] = out[:, 0:FW]
    or_ref[...] = out[:, FW:DH]


def _dense_layer(hl, hr, parts, inv2d, Ws, Wn, b2d):
    bm = 2000
    return pl.pallas_call(
        _dense_body,
        grid=(NN // bm,),
        in_specs=[
            pl.BlockSpec((bm, FW), lambda i: (i, 0)),
            pl.BlockSpec((bm, FW), lambda i: (i, 0)),
            pl.BlockSpec((NCORES, bm, FW), lambda i: (0, i, 0)),
            pl.BlockSpec((bm, 1), lambda i: (i, 0)),
            pl.BlockSpec((DH, DH), lambda i: (0, 0)),
            pl.BlockSpec((DH, DH), lambda i: (0, 0)),
            pl.BlockSpec((1, DH), lambda i: (0, 0)),
        ],
        out_specs=[pl.BlockSpec((bm, FW), lambda i: (i, 0)),
                   pl.BlockSpec((bm, FW), lambda i: (i, 0))],
        out_shape=[jax.ShapeDtypeStruct((NN, FW), F32),
                   jax.ShapeDtypeStruct((NN, FW), F32)],
    )(hl, hr, parts, inv2d, Ws, Wn, b2d)


def _head_body(hl_ref, hr_ref, c_ref, w3s, w3n, b3_, p1w, p1b, p2w, p2b,
               p3w, p3b, o_ref, accl, accr):
    i = pl.program_id(0)

    @pl.when(i == 0)
    def _():
        accl[...] = jnp.zeros_like(accl)
        accr[...] = jnp.zeros_like(accr)

    hbl = hl_ref[...]
    hbr = hr_ref[...]
    cb = c_ref[...]
    accl[0:1, :] += jnp.sum(hbl, axis=0, keepdims=True)
    accl[1:2, :] += jnp.sum(hbl * cb, axis=0, keepdims=True)
    accr[0:1, :] += jnp.sum(hbr, axis=0, keepdims=True)
    accr[1:2, :] += jnp.sum(hbr * cb, axis=0, keepdims=True)

    @pl.when(i == pl.num_programs(0) - 1)
    def _():
        al = accl[...] * (1.0 / NN)
        ar = accr[...] * (1.0 / NN)
        emb = (jnp.dot(al[0:1, :], w3s[0:FW, :], preferred_element_type=F32)
               + jnp.dot(ar[0:1, :], w3s[FW:DH, :],
                         preferred_element_type=F32)
               + jnp.dot(al[1:2, :], w3n[0:FW, :], preferred_element_type=F32)
               + jnp.dot(ar[1:2, :], w3n[FW:DH, :],
                         preferred_element_type=F32)
               + b3_[...])
        z = jnp.maximum(jnp.dot(emb, p1w[...], preferred_element_type=F32)
                        + p1b[...], 0.0)
        z = jnp.maximum(jnp.dot(z, p2w[...], preferred_element_type=F32)
                        + p2b[...], 0.0)
        o_ref[...] = (jnp.dot(z, p3w[...], preferred_element_type=F32)
                      + p3b[...]) * 2.0


def _head(hl, hr, c2d, W3s, W3n, b3, P1W, P1b, P2W, P2b, P3W, P3b):
    bm = 2000

    def full(shp):
        return pl.BlockSpec(shp, lambda i: tuple(0 for _ in shp))

    return pl.pallas_call(
        _head_body,
        grid=(NN // bm,),
        in_specs=[
            pl.BlockSpec((bm, FW), lambda i: (i, 0)),
            pl.BlockSpec((bm, FW), lambda i: (i, 0)),
            pl.BlockSpec((bm, 1), lambda i: (i, 0)),
            full((DH, 64)), full((DH, 64)), full((1, 64)),
            full((64, 64)), full((1, 64)),
            full((64, 64)), full((1, 64)),
            full((64, 512)), full((1, 512)),
        ],
        out_specs=full((1, 512)),
        out_shape=jax.ShapeDtypeStruct((1, 512), F32),
        scratch_shapes=[pltpu.VMEM((8, FW), F32), pltpu.VMEM((8, FW), F32)],
    )(hl, hr, c2d, W3s, W3n, b3, P1W, P1b, P2W, P2b, P3W, P3b)


def kernel(x, edge_index, W1_self, W1_neigh, b1, W2_self, W2_neigh, b2,
           W3_self, W3_neigh, b3, P1_W, P1_b, P2_W, P2_b, P3_W, P3_b):
    src = edge_index[0]
    dst = edge_index[1]

    npad_e = EP - EE
    ar = jnp.arange(npad_e, dtype=jnp.int32)
    srcp = jnp.concatenate([src, ar % NN]).reshape(NSUB, CPW, CHUNK)
    dstp = jnp.concatenate([dst, NN + (ar % 16)]).reshape(NSUB, CPW, CHUNK)

    zrows = jnp.zeros((NPAD // NSUB, FW), F32)
    zstat = jnp.zeros((NNP,), F32)

    xl = x[:, 0:FW]
    xr = x[:, FW:DH]
    p1, invdeg = _agg_deg_call(xl, xr, srcp, dstp, zrows, zstat)
    inv2d = invdeg[:NN].reshape(NN, 1)
    h1l, h1r = _dense_layer(xl, xr, p1, inv2d,
                            W1_self, W1_neigh, b1.reshape(1, DH))
    p2, c = _agg_c_call(h1l, h1r, srcp, dstp, zrows, invdeg)
    c2d = c[:NN].reshape(NN, 1)
    h2l, h2r = _dense_layer(h1l, h1r, p2, inv2d,
                            W2_self, W2_neigh, b2.reshape(1, DH))

    out = _head(h2l, h2r, c2d, W3_self, W3_neigh, b3.reshape(1, 64),
                P1_W, P1_b.reshape(1, 64), P2_W, P2_b.reshape(1, 64),
                P3_W, P3_b.reshape(1, 512))
    return out[0]


# speedup vs baseline: 1.2025x; 1.2025x over previous
import functools

import jax
import jax.numpy as jnp
from jax import lax
from jax.experimental import pallas as pl
from jax.experimental.pallas import tpu as pltpu
from jax.experimental.pallas import tpu_sc as plsc

NN = 10000
EE = 320000
DH = 128
FW = 64
NPAD = 10112
NCORES = 2
NSUB = 16
NW = NCORES * NSUB
CHUNK = 128
CPW = 160
EPW = CPW * CHUNK
EP = NSUB * EPW
EPT = EE // NSUB
F32 = jnp.float32

_mesh = plsc.VectorSubcoreMesh(core_axis_name="c", subcore_axis_name="s")


NNP = 10240
RPT = NNP // NSUB


def _hist_combine(hist_v, slots, acc_v, tmp_v, ci, si):
    base = si * RPT

    @pl.when(ci == 0)
    def _():
        pltpu.sync_copy(hist_v, slots.at[si])
    plsc.subcore_barrier()

    @pl.when(ci == 0)
    def _():
        for t in range(NSUB):
            pltpu.sync_copy(slots.at[t].at[pl.ds(base, RPT)], tmp_v)
            if t == 0:
                @pl.loop(0, RPT // 16)
                def _(i):
                    acc_v[pl.ds(i * 16, 16)] = tmp_v[pl.ds(i * 16, 16)]
            else:
                @pl.loop(0, RPT // 16)
                def _(i):
                    acc_v[pl.ds(i * 16, 16)] = (acc_v[pl.ds(i * 16, 16)]
                                                + tmp_v[pl.ds(i * 16, 16)])


def _make_agg_body(mode):
    def body(hl_hbm, hr_hbm, src_hbm, dst_hbm, z_hbm, inv_hbm,
             out_hbm, stat_out,
             src_v, dst_v, rows0, rows1, hist_v, stat_v,
             acc_v, tmp_v, acc, slots, semg0, semg1):
        ci = lax.axis_index("c")
        si = lax.axis_index("s")
        rows_per_tile = NPAD // NSUB
        base = si * rows_per_tile
        zeros16 = jnp.zeros((16,), F32)
        ones16 = jnp.ones((16,), F32)

        pltpu.sync_copy(src_hbm.at[si], src_v)
        pltpu.sync_copy(dst_hbm.at[si], dst_v)
        pltpu.sync_copy(z_hbm, acc.at[pl.ds(base, rows_per_tile)])

        @pl.when(ci == 0)
        def _():
            if mode == "c":
                pltpu.sync_copy(inv_hbm, stat_v)
                @pl.loop(0, (NNP - NN) // 16)
                def _(i):
                    stat_v[pl.ds(NN + i * 16, 16)] = zeros16

            @pl.loop(0, NNP // 16)
            def _(i):
                hist_v[pl.ds(i * 16, 16)] = zeros16

        plsc.subcore_barrier()

        def stat_chunk(k):
            for m in range(CHUNK // 16):
                d16 = dst_v[k, pl.ds(m * 16, 16)]
                if mode == "deg":
                    plsc.addupdate_scatter(hist_v, [d16], ones16)
                else:
                    s16 = src_v[k, pl.ds(m * 16, 16)]
                    g = plsc.load_gather(stat_v, [d16])
                    plsc.addupdate_scatter(hist_v, [s16], g)

        bufs = [rows0, rows1]
        gsems = [semg0, semg1]
        nbuf = 2

        def pipeline(h_hbm, with_stats):
            for b in range(nbuf):
                pltpu.async_copy(h_hbm.at[src_v.at[b]], bufs[b], gsems[b])

            @pl.loop(0, CPW, step=nbuf)
            def _(j):
                for b in range(nbuf):
                    k = j + b
                    if with_stats:
                        stat_chunk(k)
                    pltpu.make_async_copy(h_hbm.at[src_v.at[0]], bufs[b],
                                          gsems[b]).wait()
                    pltpu.sync_copy(bufs[b], acc.at[dst_v.at[k]], add=True)

                    @pl.when(k + nbuf < CPW)
                    def _():
                        pltpu.async_copy(h_hbm.at[src_v.at[k + nbuf]],
                                         bufs[b], gsems[b])

        @pl.when(ci == 0)
        def _():
            pipeline(hl_hbm, True)

        @pl.when(ci == 1)
        def _():
            pipeline(hr_hbm, False)

        plsc.subcore_barrier()
        pltpu.sync_copy(acc.at[pl.ds(base, rows_per_tile)],
                        out_hbm.at[ci].at[pl.ds(base, rows_per_tile)])

        _hist_combine(hist_v, slots, acc_v, tmp_v, ci, si)

        @pl.when(ci == 0)
        def _():
            if mode == "deg":
                @pl.loop(0, RPT // 16)
                def _(i):
                    d = acc_v[pl.ds(i * 16, 16)]
                    acc_v[pl.ds(i * 16, 16)] = 1.0 / jnp.maximum(d, 1.0)

            pltpu.sync_copy(acc_v, stat_out.at[pl.ds(si * RPT, RPT)])

    return body


def _make_agg_call(mode):
    return pl.kernel(
        _make_agg_body(mode),
        out_type=(jax.ShapeDtypeStruct((NCORES, NPAD, FW), F32),
                  jax.ShapeDtypeStruct((NNP,), F32)),
        mesh=_mesh,
        scratch_types=[
            pltpu.VMEM((CPW, CHUNK), jnp.int32),
            pltpu.VMEM((CPW, CHUNK), jnp.int32),
            pltpu.VMEM((CHUNK, FW), F32),
            pltpu.VMEM((CHUNK, FW), F32),
            pltpu.VMEM((NNP,), F32),
            pltpu.VMEM((NNP,), F32),
            pltpu.VMEM((RPT,), F32),
            pltpu.VMEM((RPT,), F32),
            pltpu.VMEM_SHARED((NPAD, FW), F32),
            pltpu.VMEM_SHARED((NSUB, NNP), F32),
        ] + [pltpu.SemaphoreType.DMA] * 2,
        compiler_params=pltpu.CompilerParams(needs_layout_passes=False,
                                             use_tc_tiling_on_sc=False),
    )


_agg_deg_call = _make_agg_call("deg")
_agg_c_call = _make_agg_call("c")


def _dense_body(hl_ref, hr_ref, p_ref, inv_ref, ws_ref, wn_ref,
                b_ref, ol_ref, or_ref):
    inv = inv_ref[...]
    agg_l = p_ref[0] * inv
    agg_r = p_ref[1] * inv
    ws = ws_ref[...]
    wn = wn_ref[...]
    out = (jnp.dot(hl_ref[...], ws[0:FW, :], preferred_element_type=F32)
           + jnp.dot(hr_ref[...], ws[FW:DH, :], preferred_element_type=F32)
           + jnp.dot(agg_l, wn[0:FW, :], preferred_element_type=F32)
           + jnp.dot(agg_r, wn[FW:DH, :], preferred_element_type=F32)
           + b_ref[...])
    out = jnp.maximum(out, 0.0)
    ol_ref[...] = out[:, 0:FW]
    or_ref[...] = out[:, FW:DH]


def _dense_layer(hl, hr, parts, inv2d, Ws, Wn, b2d):
    bm = 2000
    return pl.pallas_call(
        _dense_body,
        grid=(NN // bm,),
        in_specs=[
            pl.BlockSpec((bm, FW), lambda i: (i, 0)),
            pl.BlockSpec((bm, FW), lambda i: (i, 0)),
            pl.BlockSpec((NCORES, bm, FW), lambda i: (0, i, 0)),
            pl.BlockSpec((bm, 1), lambda i: (i, 0)),
            pl.BlockSpec((DH, DH), lambda i: (0, 0)),
            pl.BlockSpec((DH, DH), lambda i: (0, 0)),
            pl.BlockSpec((1, DH), lambda i: (0, 0)),
        ],
        out_specs=[pl.BlockSpec((bm, FW), lambda i: (i, 0)),
                   pl.BlockSpec((bm, FW), lambda i: (i, 0))],
        out_shape=[jax.ShapeDtypeStruct((NN, FW), F32),
                   jax.ShapeDtypeStruct((NN, FW), F32)],
    )(hl, hr, parts, inv2d, Ws, Wn, b2d)


def _l2head_body(hl_ref, hr_ref, p_ref, inv_ref, c_ref, w2s, w2n, b2_,
                 w3s, w3n, b3_, p1w, p1b, p2w, p2b, p3w, p3b, o_ref, acc):
    i = pl.program_id(0)

    @pl.when(i == 0)
    def _():
        acc[...] = jnp.zeros_like(acc)

    inv = inv_ref[...]
    agg_l = p_ref[0] * inv
    agg_r = p_ref[1] * inv
    ws = w2s[...]
    wn = w2n[...]
    h2 = (jnp.dot(hl_ref[...], ws[0:FW, :], preferred_element_type=F32)
          + jnp.dot(hr_ref[...], ws[FW:DH, :], preferred_element_type=F32)
          + jnp.dot(agg_l, wn[0:FW, :], preferred_element_type=F32)
          + jnp.dot(agg_r, wn[FW:DH, :], preferred_element_type=F32)
          + b2_[...])
    h2 = jnp.maximum(h2, 0.0)
    cb = c_ref[...]
    acc[0:1, :] += jnp.sum(h2, axis=0, keepdims=True)
    acc[1:2, :] += jnp.sum(h2 * cb, axis=0, keepdims=True)

    @pl.when(i == pl.num_programs(0) - 1)
    def _():
        a = acc[...] * (1.0 / NN)
        emb = (jnp.dot(a[0:1, :], w3s[...], preferred_element_type=F32)
               + jnp.dot(a[1:2, :], w3n[...], preferred_element_type=F32)
               + b3_[...])
        z = jnp.maximum(jnp.dot(emb, p1w[...], preferred_element_type=F32)
                        + p1b[...], 0.0)
        z = jnp.maximum(jnp.dot(z, p2w[...], preferred_element_type=F32)
                        + p2b[...], 0.0)
        o_ref[...] = (jnp.dot(z, p3w[...], preferred_element_type=F32)
                      + p3b[...]) * 2.0


def _l2head(hl, hr, parts, inv2d, c2d, W2s, W2n, b2d,
            W3s, W3n, b3, P1W, P1b, P2W, P2b, P3W, P3b):
    bm = 2000

    def full(shp):
        return pl.BlockSpec(shp, lambda i: tuple(0 for _ in shp))

    return pl.pallas_call(
        _l2head_body,
        grid=(NN // bm,),
        in_specs=[
            pl.BlockSpec((bm, FW), lambda i: (i, 0)),
            pl.BlockSpec((bm, FW), lambda i: (i, 0)),
            pl.BlockSpec((NCORES, bm, FW), lambda i: (0, i, 0)),
            pl.BlockSpec((bm, 1), lambda i: (i, 0)),
            pl.BlockSpec((bm, 1), lambda i: (i, 0)),
            full((DH, DH)), full((DH, DH)), full((1, DH)),
            full((DH, 64)), full((DH, 64)), full((1, 64)),
            full((64, 64)), full((1, 64)),
            full((64, 64)), full((1, 64)),
            full((64, 512)), full((1, 512)),
        ],
        out_specs=full((1, 512)),
        out_shape=jax.ShapeDtypeStruct((1, 512), F32),
        scratch_shapes=[pltpu.VMEM((8, DH), F32)],
    )(hl, hr, parts, inv2d, c2d, W2s, W2n, b2d,
      W3s, W3n, b3, P1W, P1b, P2W, P2b, P3W, P3b)


def kernel(x, edge_index, W1_self, W1_neigh, b1, W2_self, W2_neigh, b2,
           W3_self, W3_neigh, b3, P1_W, P1_b, P2_W, P2_b, P3_W, P3_b):
    src = edge_index[0]
    dst = edge_index[1]

    npad_e = EP - EE
    ar = jnp.arange(npad_e, dtype=jnp.int32)
    srcp = jnp.concatenate([src, ar % NN]).reshape(NSUB, CPW, CHUNK)
    dstp = jnp.concatenate([dst, NN + (ar % 16)]).reshape(NSUB, CPW, CHUNK)

    zrows = jnp.zeros((NPAD // NSUB, FW), F32)
    zstat = jnp.zeros((NNP,), F32)

    xl = x[:, 0:FW]
    xr = x[:, FW:DH]
    p1, invdeg = _agg_deg_call(xl, xr, srcp, dstp, zrows, zstat)
    inv2d = invdeg[:NN].reshape(NN, 1)
    h1l, h1r = _dense_layer(xl, xr, p1, inv2d,
                            W1_self, W1_neigh, b1.reshape(1, DH))
    p2, c = _agg_c_call(h1l, h1r, srcp, dstp, zrows, invdeg)
    c2d = c[:NN].reshape(NN, 1)

    out = _l2head(h1l, h1r, p2, inv2d, c2d,
                  W2_self, W2_neigh, b2.reshape(1, DH),
                  W3_self, W3_neigh, b3.reshape(1, 64),
                  P1_W, P1_b.reshape(1, 64), P2_W, P2_b.reshape(1, 64),
                  P3_W, P3_b.reshape(1, 512))
    return out[0]


# speedup vs baseline: 1.2128x; 1.0085x over previous
import functools

import jax
import jax.numpy as jnp
from jax import lax
from jax.experimental import pallas as pl
from jax.experimental.pallas import tpu as pltpu
from jax.experimental.pallas import tpu_sc as plsc

NN = 10000
EE = 320000
DH = 128
FW = 64
NPAD = 10112
NCORES = 2
NSUB = 16
NW = NCORES * NSUB
CHUNK = 128
NCH = EE // CHUNK
CPA = 157
F32 = jnp.float32

_mesh = plsc.VectorSubcoreMesh(core_axis_name="c", subcore_axis_name="s")


NNP = 10240
RPT = NNP // NSUB


def _hist_combine(hist_v, slots, acc_v, tmp_v, ci, si):
    base = si * RPT

    @pl.when(ci == 0)
    def _():
        pltpu.sync_copy(hist_v, slots.at[si])
    plsc.subcore_barrier()

    @pl.when(ci == 0)
    def _():
        for t in range(NSUB):
            pltpu.sync_copy(slots.at[t].at[pl.ds(base, RPT)], tmp_v)
            if t == 0:
                @pl.loop(0, RPT // 16)
                def _(i):
                    acc_v[pl.ds(i * 16, 16)] = tmp_v[pl.ds(i * 16, 16)]
            else:
                @pl.loop(0, RPT // 16)
                def _(i):
                    acc_v[pl.ds(i * 16, 16)] = (acc_v[pl.ds(i * 16, 16)]
                                                + tmp_v[pl.ds(i * 16, 16)])


def _make_agg_body(mode):
    def body(hl_hbm, hr_hbm, src_hbm, dst_hbm, z_hbm, inv_hbm,
             out_hbm, stat_out,
             src_v, dst_v, rows0, rows1, hist_v, stat_v,
             acc_v, tmp_v, acc, slots, semg0, semg1):
        ci = lax.axis_index("c")
        si = lax.axis_index("s")
        rows_per_tile = NPAD // NSUB
        base = si * rows_per_tile
        zeros16 = jnp.zeros((16,), F32)
        ones16 = jnp.ones((16,), F32)
        start = si * 156 + jnp.minimum(si, 4)
        nch = jnp.where(si < 4, 157, 156)

        @pl.when(si < 4)
        def _():
            pltpu.sync_copy(src_hbm.at[pl.ds(start, 157)], src_v)
            pltpu.sync_copy(dst_hbm.at[pl.ds(start, 157)], dst_v)

        @pl.when(si >= 4)
        def _():
            pltpu.sync_copy(src_hbm.at[pl.ds(start, 156)],
                            src_v.at[pl.ds(0, 156)])
            pltpu.sync_copy(dst_hbm.at[pl.ds(start, 156)],
                            dst_v.at[pl.ds(0, 156)])

        pltpu.sync_copy(z_hbm, acc.at[pl.ds(base, rows_per_tile)])

        @pl.when(ci == 0)
        def _():
            if mode == "c":
                pltpu.sync_copy(inv_hbm, stat_v)
                @pl.loop(0, (NNP - NN) // 16)
                def _(i):
                    stat_v[pl.ds(NN + i * 16, 16)] = zeros16

            @pl.loop(0, NNP // 16)
            def _(i):
                hist_v[pl.ds(i * 16, 16)] = zeros16

        plsc.subcore_barrier()

        def stat_chunk(k):
            for m in range(CHUNK // 16):
                d16 = dst_v[k, pl.ds(m * 16, 16)]
                if mode == "deg":
                    plsc.addupdate_scatter(hist_v, [d16], ones16)
                else:
                    s16 = src_v[k, pl.ds(m * 16, 16)]
                    g = plsc.load_gather(stat_v, [d16])
                    plsc.addupdate_scatter(hist_v, [s16], g)

        bufs = [rows0, rows1]
        gsems = [semg0, semg1]
        nbuf = 2

        def pipeline(h_hbm, with_stats):
            for b in range(nbuf):
                pltpu.async_copy(h_hbm.at[src_v.at[b]], bufs[b], gsems[b])

            @pl.loop(0, 156, step=nbuf)
            def _(j):
                for b in range(nbuf):
                    k = j + b
                    if with_stats:
                        stat_chunk(k)
                    pltpu.make_async_copy(h_hbm.at[src_v.at[0]], bufs[b],
                                          gsems[b]).wait()
                    pltpu.sync_copy(bufs[b], acc.at[dst_v.at[k]], add=True)

                    @pl.when(k + nbuf < nch)
                    def _():
                        pltpu.async_copy(h_hbm.at[src_v.at[k + nbuf]],
                                         bufs[b], gsems[b])

            @pl.when(si < 4)
            def _():
                if with_stats:
                    stat_chunk(156)
                pltpu.make_async_copy(h_hbm.at[src_v.at[0]], bufs[0],
                                      gsems[0]).wait()
                pltpu.sync_copy(bufs[0], acc.at[dst_v.at[156]], add=True)

        @pl.when(ci == 0)
        def _():
            pipeline(hl_hbm, True)

        @pl.when(ci == 1)
        def _():
            pipeline(hr_hbm, False)

        plsc.subcore_barrier()
        pltpu.sync_copy(acc.at[pl.ds(base, rows_per_tile)],
                        out_hbm.at[ci].at[pl.ds(base, rows_per_tile)])

        _hist_combine(hist_v, slots, acc_v, tmp_v, ci, si)

        @pl.when(ci == 0)
        def _():
            if mode == "deg":
                @pl.loop(0, RPT // 16)
                def _(i):
                    d = acc_v[pl.ds(i * 16, 16)]
                    acc_v[pl.ds(i * 16, 16)] = 1.0 / jnp.maximum(d, 1.0)

            pltpu.sync_copy(acc_v, stat_out.at[pl.ds(si * RPT, RPT)])

    return body


def _make_agg_call(mode):
    return pl.kernel(
        _make_agg_body(mode),
        out_type=(jax.ShapeDtypeStruct((NCORES, NPAD, FW), F32),
                  jax.ShapeDtypeStruct((NNP,), F32)),
        mesh=_mesh,
        scratch_types=[
            pltpu.VMEM((CPA, CHUNK), jnp.int32),
            pltpu.VMEM((CPA, CHUNK), jnp.int32),
            pltpu.VMEM((CHUNK, FW), F32),
            pltpu.VMEM((CHUNK, FW), F32),
            pltpu.VMEM((NNP,), F32),
            pltpu.VMEM((NNP,), F32),
            pltpu.VMEM((RPT,), F32),
            pltpu.VMEM((RPT,), F32),
            pltpu.VMEM_SHARED((NPAD, FW), F32),
            pltpu.VMEM_SHARED((NSUB, NNP), F32),
        ] + [pltpu.SemaphoreType.DMA] * 2,
        compiler_params=pltpu.CompilerParams(needs_layout_passes=False,
                                             use_tc_tiling_on_sc=False),
    )


_agg_deg_call = _make_agg_call("deg")
_agg_c_call = _make_agg_call("c")


def _dense_body(hl_ref, hr_ref, p_ref, inv_ref, ws_ref, wn_ref,
                b_ref, ol_ref, or_ref):
    inv = inv_ref[...]
    agg_l = p_ref[0] * inv
    agg_r = p_ref[1] * inv
    ws = ws_ref[...]
    wn = wn_ref[...]
    out = (jnp.dot(hl_ref[...], ws[0:FW, :], preferred_element_type=F32)
           + jnp.dot(hr_ref[...], ws[FW:DH, :], preferred_element_type=F32)
           + jnp.dot(agg_l, wn[0:FW, :], preferred_element_type=F32)
           + jnp.dot(agg_r, wn[FW:DH, :], preferred_element_type=F32)
           + b_ref[...])
    out = jnp.maximum(out, 0.0)
    ol_ref[...] = out[:, 0:FW]
    or_ref[...] = out[:, FW:DH]


def _dense_layer(hl, hr, parts, inv2d, Ws, Wn, b2d):
    bm = 2000
    return pl.pallas_call(
        _dense_body,
        grid=(NN // bm,),
        in_specs=[
            pl.BlockSpec((bm, FW), lambda i: (i, 0)),
            pl.BlockSpec((bm, FW), lambda i: (i, 0)),
            pl.BlockSpec((NCORES, bm, FW), lambda i: (0, i, 0)),
            pl.BlockSpec((bm, 1), lambda i: (i, 0)),
            pl.BlockSpec((DH, DH), lambda i: (0, 0)),
            pl.BlockSpec((DH, DH), lambda i: (0, 0)),
            pl.BlockSpec((1, DH), lambda i: (0, 0)),
        ],
        out_specs=[pl.BlockSpec((bm, FW), lambda i: (i, 0)),
                   pl.BlockSpec((bm, FW), lambda i: (i, 0))],
        out_shape=[jax.ShapeDtypeStruct((NN, FW), F32),
                   jax.ShapeDtypeStruct((NN, FW), F32)],
    )(hl, hr, parts, inv2d, Ws, Wn, b2d)


def _l2head_body(hl_ref, hr_ref, p_ref, inv_ref, c_ref, w2s, w2n, b2_,
                 w3s, w3n, b3_, p1w, p1b, p2w, p2b, p3w, p3b, o_ref, acc):
    i = pl.program_id(0)

    @pl.when(i == 0)
    def _():
        acc[...] = jnp.zeros_like(acc)

    inv = inv_ref[...]
    agg_l = p_ref[0] * inv
    agg_r = p_ref[1] * inv
    ws = w2s[...]
    wn = w2n[...]
    h2 = (jnp.dot(hl_ref[...], ws[0:FW, :], preferred_element_type=F32)
          + jnp.dot(hr_ref[...], ws[FW:DH, :], preferred_element_type=F32)
          + jnp.dot(agg_l, wn[0:FW, :], preferred_element_type=F32)
          + jnp.dot(agg_r, wn[FW:DH, :], preferred_element_type=F32)
          + b2_[...])
    h2 = jnp.maximum(h2, 0.0)
    cb = c_ref[...]
    acc[0:1, :] += jnp.sum(h2, axis=0, keepdims=True)
    acc[1:2, :] += jnp.sum(h2 * cb, axis=0, keepdims=True)

    @pl.when(i == pl.num_programs(0) - 1)
    def _():
        a = acc[...] * (1.0 / NN)
        emb = (jnp.dot(a[0:1, :], w3s[...], preferred_element_type=F32)
               + jnp.dot(a[1:2, :], w3n[...], preferred_element_type=F32)
               + b3_[...])
        z = jnp.maximum(jnp.dot(emb, p1w[...], preferred_element_type=F32)
                        + p1b[...], 0.0)
        z = jnp.maximum(jnp.dot(z, p2w[...], preferred_element_type=F32)
                        + p2b[...], 0.0)
        o_ref[...] = (jnp.dot(z, p3w[...], preferred_element_type=F32)
                      + p3b[...]) * 2.0


def _l2head(hl, hr, parts, inv2d, c2d, W2s, W2n, b2d,
            W3s, W3n, b3, P1W, P1b, P2W, P2b, P3W, P3b):
    bm = 2000

    def full(shp):
        return pl.BlockSpec(shp, lambda i: tuple(0 for _ in shp))

    return pl.pallas_call(
        _l2head_body,
        grid=(NN // bm,),
        in_specs=[
            pl.BlockSpec((bm, FW), lambda i: (i, 0)),
            pl.BlockSpec((bm, FW), lambda i: (i, 0)),
            pl.BlockSpec((NCORES, bm, FW), lambda i: (0, i, 0)),
            pl.BlockSpec((bm, 1), lambda i: (i, 0)),
            pl.BlockSpec((bm, 1), lambda i: (i, 0)),
            full((DH, DH)), full((DH, DH)), full((1, DH)),
            full((DH, 64)), full((DH, 64)), full((1, 64)),
            full((64, 64)), full((1, 64)),
            full((64, 64)), full((1, 64)),
            full((64, 512)), full((1, 512)),
        ],
        out_specs=full((1, 512)),
        out_shape=jax.ShapeDtypeStruct((1, 512), F32),
        scratch_shapes=[pltpu.VMEM((8, DH), F32)],
    )(hl, hr, parts, inv2d, c2d, W2s, W2n, b2d,
      W3s, W3n, b3, P1W, P1b, P2W, P2b, P3W, P3b)


def kernel(x, edge_index, W1_self, W1_neigh, b1, W2_self, W2_neigh, b2,
           W3_self, W3_neigh, b3, P1_W, P1_b, P2_W, P2_b, P3_W, P3_b):
    srcp = edge_index[0].reshape(NCH, CHUNK)
    dstp = edge_index[1].reshape(NCH, CHUNK)

    zrows = jnp.zeros((NPAD // NSUB, FW), F32)
    zstat = jnp.zeros((NNP,), F32)

    xl = x[:, 0:FW]
    xr = x[:, FW:DH]
    p1, invdeg = _agg_deg_call(xl, xr, srcp, dstp, zrows, zstat)
    inv2d = invdeg[:NN].reshape(NN, 1)
    h1l, h1r = _dense_layer(xl, xr, p1, inv2d,
                            W1_self, W1_neigh, b1.reshape(1, DH))
    p2, c = _agg_c_call(h1l, h1r, srcp, dstp, zrows, invdeg)
    c2d = c[:NN].reshape(NN, 1)

    out = _l2head(h1l, h1r, p2, inv2d, c2d,
                  W2_self, W2_neigh, b2.reshape(1, DH),
                  W3_self, W3_neigh, b3.reshape(1, 64),
                  P1_W, P1_b.reshape(1, 64), P2_W, P2_b.reshape(1, 64),
                  P3_W, P3_b.reshape(1, 512))
    return out[0]
